# baseline (speedup 1.0000x reference)
import jax
import jax.numpy as jnp
from jax.experimental import pallas as pl
from jax.experimental.pallas import tpu as pltpu

D = 768
RANK = 192
M_BLK = 512
CH = 2
NBUF = 4


def _fused_lowrank_kernel(x_hbm, u_ref, v_ref, o_hbm,
                          x_vmem, o_vmem,
                          in_sem_a, in_sem_b, out_sem_a, out_sem_b):
    in_sems = (in_sem_a, in_sem_b)
    out_sems = (out_sem_a, out_sem_b)
    m = x_hbm.shape[0]
    tpc = m // M_BLK // CH

    def in_copy(c, k, slot):
        tile = k * CH + c
        return pltpu.make_async_copy(
            x_hbm.at[pl.ds(tile * M_BLK, M_BLK), :],
            x_vmem.at[c * NBUF + slot],
            in_sems[c].at[slot])

    def out_copy(c, k, slot):
        tile = k * CH + c
        return pltpu.make_async_copy(
            o_vmem.at[c * NBUF + slot],
            o_hbm.at[pl.ds(tile * M_BLK, M_BLK), :],
            out_sems[c].at[slot])

    for c in range(CH):
        for k in range(NBUF - 1):
            in_copy(c, k, k).start(priority=c)

    def loop(k, carry):
        slot = jax.lax.rem(k, NBUF)
        nxt = k + NBUF - 1
        nxt_slot = jax.lax.rem(nxt, NBUF)
        for c in range(CH):
            @pl.when(nxt < tpc)
            def _():
                in_copy(c, nxt, nxt_slot).start(priority=c)

            in_copy(c, k, slot).wait()

            @pl.when(k >= NBUF)
            def _():
                out_copy(c, k - NBUF, slot).wait()

            h = jnp.dot(x_vmem[c * NBUF + slot], u_ref[...],
                        preferred_element_type=jnp.float32)
            o_vmem[c * NBUF + slot] = jnp.dot(
                h, v_ref[...], preferred_element_type=jnp.float32)
            out_copy(c, k, slot).start(priority=c)
        return carry

    jax.lax.fori_loop(0, tpc, loop, 0)

    for c in range(CH):
        for k in range(tpc - NBUF, tpc):
            out_copy(c, k, k % NBUF).wait()


def kernel(x, U, V):
    b, s, d = x.shape
    m = b * s
    x2 = x.reshape(m, d)
    out = pl.pallas_call(
        _fused_lowrank_kernel,
        in_specs=[
            pl.BlockSpec(memory_space=pltpu.MemorySpace.HBM),
            pl.BlockSpec(memory_space=pltpu.MemorySpace.VMEM),
            pl.BlockSpec(memory_space=pltpu.MemorySpace.VMEM),
        ],
        out_specs=pl.BlockSpec(memory_space=pltpu.MemorySpace.HBM),
        out_shape=jax.ShapeDtypeStruct((m, d), x.dtype),
        scratch_shapes=[
            pltpu.VMEM((CH * NBUF, M_BLK, D), jnp.float32),
            pltpu.VMEM((CH * NBUF, M_BLK, D), jnp.float32),
            pltpu.SemaphoreType.DMA((NBUF,)),
            pltpu.SemaphoreType.DMA((NBUF,)),
            pltpu.SemaphoreType.DMA((NBUF,)),
            pltpu.SemaphoreType.DMA((NBUF,)),
        ],
    )(x2, U, V)
    return out.reshape(b, s, d)


# speedup vs baseline: 1.0738x; 1.0738x over previous
import jax
import jax.numpy as jnp
from jax.experimental import pallas as pl
from jax.experimental.pallas import tpu as pltpu

D = 768
RANK = 192
M_BLK = 1024
NBUF = 3


def _fused_lowrank_kernel(x_hbm, u_ref, v_ref, o_hbm,
                          w_vmem, x_vmem, o_vmem, in_sems, out_sems):
    m = x_hbm.shape[0]
    num = m // M_BLK

    def in_copy(i, slot):
        return pltpu.make_async_copy(
            x_hbm.at[pl.ds(i * M_BLK, M_BLK), :], x_vmem.at[slot],
            in_sems.at[slot])

    def out_copy(i, slot):
        return pltpu.make_async_copy(
            o_vmem.at[slot], o_hbm.at[pl.ds(i * M_BLK, M_BLK), :],
            out_sems.at[slot])

    for k in range(NBUF - 1):
        in_copy(k, k).start()

    w_vmem[...] = jnp.dot(u_ref[...], v_ref[...],
                          preferred_element_type=jnp.float32)

    def loop(i, carry):
        slot = jax.lax.rem(i, NBUF)
        nxt = i + NBUF - 1

        @pl.when(nxt < num)
        def _():
            in_copy(nxt, jax.lax.rem(nxt, NBUF)).start()

        in_copy(i, slot).wait()

        @pl.when(i >= NBUF)
        def _():
            out_copy(i - NBUF, slot).wait()

        o_vmem[slot] = jnp.dot(x_vmem[slot], w_vmem[...],
                               preferred_element_type=jnp.float32)
        out_copy(i, slot).start()
        return carry

    jax.lax.fori_loop(0, num, loop, 0)

    for i in range(num - NBUF, num):
        out_copy(i, i % NBUF).wait()


def kernel(x, U, V):
    b, s, d = x.shape
    m = b * s
    x2 = x.reshape(m, d)
    out = pl.pallas_call(
        _fused_lowrank_kernel,
        in_specs=[
            pl.BlockSpec(memory_space=pltpu.MemorySpace.HBM),
            pl.BlockSpec(memory_space=pltpu.MemorySpace.VMEM),
            pl.BlockSpec(memory_space=pltpu.MemorySpace.VMEM),
        ],
        out_specs=pl.BlockSpec(memory_space=pltpu.MemorySpace.HBM),
        out_shape=jax.ShapeDtypeStruct((m, d), x.dtype),
        scratch_shapes=[
            pltpu.VMEM((D, D), jnp.float32),
            pltpu.VMEM((NBUF, M_BLK, D), jnp.float32),
            pltpu.VMEM((NBUF, M_BLK, D), jnp.float32),
            pltpu.SemaphoreType.DMA((NBUF,)),
            pltpu.SemaphoreType.DMA((NBUF,)),
        ],
    )(x2, U, V)
    return out.reshape(b, s, d)


# speedup vs baseline: 1.1356x; 1.0576x over previous
import jax
import jax.numpy as jnp
from jax.experimental import pallas as pl
from jax.experimental.pallas import tpu as pltpu

D = 768
RANK = 192
M_BLK = 2048
NBUF = 3


def _fused_lowrank_kernel(x_hbm, u_ref, v_ref, o_hbm,
                          w_vmem, x_vmem, o_vmem, in_sems, out_sems):
    m = x_hbm.shape[0]
    num = m // M_BLK

    def in_copy(i, slot):
        return pltpu.make_async_copy(
            x_hbm.at[pl.ds(i * M_BLK, M_BLK), :], x_vmem.at[slot],
            in_sems.at[slot])

    def out_copy(i, slot):
        return pltpu.make_async_copy(
            o_vmem.at[slot], o_hbm.at[pl.ds(i * M_BLK, M_BLK), :],
            out_sems.at[slot])

    for k in range(NBUF - 1):
        in_copy(k, k).start()

    w_vmem[...] = jnp.dot(u_ref[...], v_ref[...],
                          preferred_element_type=jnp.float32)

    def loop(i, carry):
        slot = jax.lax.rem(i, NBUF)
        nxt = i + NBUF - 1

        @pl.when(nxt < num)
        def _():
            in_copy(nxt, jax.lax.rem(nxt, NBUF)).start()

        in_copy(i, slot).wait()

        @pl.when(i >= NBUF)
        def _():
            out_copy(i - NBUF, slot).wait()

        o_vmem[slot] = jnp.dot(x_vmem[slot], w_vmem[...],
                               preferred_element_type=jnp.float32)
        out_copy(i, slot).start()
        return carry

    jax.lax.fori_loop(0, num, loop, 0)

    for i in range(num - NBUF, num):
        out_copy(i, i % NBUF).wait()


def kernel(x, U, V):
    b, s, d = x.shape
    m = b * s
    x2 = x.reshape(m, d)
    out = pl.pallas_call(
        _fused_lowrank_kernel,
        in_specs=[
            pl.BlockSpec(memory_space=pltpu.MemorySpace.HBM),
            pl.BlockSpec(memory_space=pltpu.MemorySpace.VMEM),
            pl.BlockSpec(memory_space=pltpu.MemorySpace.VMEM),
        ],
        out_specs=pl.BlockSpec(memory_space=pltpu.MemorySpace.HBM),
        out_shape=jax.ShapeDtypeStruct((m, d), x.dtype),
        scratch_shapes=[
            pltpu.VMEM((D, D), jnp.float32),
            pltpu.VMEM((NBUF, M_BLK, D), jnp.float32),
            pltpu.VMEM((NBUF, M_BLK, D), jnp.float32),
            pltpu.SemaphoreType.DMA((NBUF,)),
            pltpu.SemaphoreType.DMA((NBUF,)),
        ],
    )(x2, U, V)
    return out.reshape(b, s, d)
